# baseline (speedup 1.0000x reference)
import jax
import jax.numpy as jnp
from jax.experimental import pallas as pl
from jax.experimental.pallas import tpu as pltpu
from jax.experimental.pallas import tpu_sc as plsc

NUM_CODES = 1024
DIM = 512
ROWS = 8192
BLK = 1024
NBLK = ROWS // BLK
BETA = 0.25

R_SUB = 64
N_CHUNK = 128
N_BLOCK = 256

N_WORKERS = 32
ROWS_PER_W = ROWS // N_WORKERS
G_CHUNK = 64
N_GCHUNK = ROWS_PER_W // G_CHUNK


def _dist_kernel(z_ref, w_ref, idx_ref, loss_ref, acc_ref, ws_ref):
    b = pl.program_id(0)

    @pl.when(b == 0)
    def _():
        w0 = w_ref[...]
        ws_ref[0, :] = jnp.sum(w0 * w0, axis=1)
        acc_ref[0] = 0.0

    z = z_ref[...]
    z2 = z + z
    w = w_ref[...]
    mm2 = [
        jax.lax.dot_general(
            z2, w[i * N_BLOCK:(i + 1) * N_BLOCK, :],
            (((1,), (1,)), ((), ())),
            preferred_element_type=jnp.float32,
            precision=jax.lax.Precision.DEFAULT,
        )
        for i in range(NUM_CODES // N_BLOCK)
    ]
    rs = jnp.sum(z * z, axis=1, keepdims=True)
    nrs = -rs
    ws = ws_ref[0, :]

    part = None
    for r in range(BLK // R_SUB):
        rsl = slice(r * R_SUB, (r + 1) * R_SUB)
        nrb = nrs[rsl]
        m = None
        cid = None
        for c in range(NUM_CODES // N_CHUNK):
            csl = slice(c * N_CHUNK, (c + 1) * N_CHUNK)
            mm_tile = mm2[(c * N_CHUNK) // N_BLOCK]
            col = (c * N_CHUNK) % N_BLOCK
            t = nrb - ws[None, csl]
            d = t + mm_tile[rsl, col:col + N_CHUNK]
            if c == 0:
                m = d
                cid = jnp.zeros((R_SUB, N_CHUNK), jnp.int32)
            else:
                cond = d > m
                m = jnp.where(cond, d, m)
                cid = jnp.where(cond, c * N_CHUNK, cid)
        M = jnp.max(m, axis=1)
        lane = jax.lax.broadcasted_iota(jnp.int32, (R_SUB, N_CHUNK), 1)
        gid = cid + lane
        idxr = jnp.min(jnp.where(m == M[:, None], gid, NUM_CODES), axis=1)
        idx_ref[0, r * R_SUB:(r + 1) * R_SUB] = idxr
        psum = jnp.sum(-M)
        part = psum if part is None else part + psum
    acc_ref[0] += part

    @pl.when(b == pl.num_programs(0) - 1)
    def _():
        loss_ref[0, 0] = acc_ref[0]


N_HALF = 1
HROWS = ROWS // N_HALF
NBLK_H = HROWS // BLK


def _distance_argmax(z_flat, W, half):
    return pl.pallas_call(
        _dist_kernel,
        grid=(NBLK_H,),
        in_specs=[
            pl.BlockSpec((BLK, DIM), lambda b: (b + half * NBLK_H, 0)),
            pl.BlockSpec((NUM_CODES, DIM), lambda b: (0, 0)),
        ],
        out_specs=[
            pl.BlockSpec((1, BLK), lambda b: (0, b)),
            pl.BlockSpec(memory_space=pltpu.SMEM),
        ],
        out_shape=[
            jax.ShapeDtypeStruct((1, HROWS), jnp.int32),
            jax.ShapeDtypeStruct((1, 1), jnp.float32),
        ],
        scratch_shapes=[
            pltpu.SMEM((1,), jnp.float32),
            pltpu.VMEM((1, NUM_CODES), jnp.float32),
        ],
    )(z_flat, W)


HROWS_PER_W = HROWS // N_WORKERS
N_GCHUNK_H = HROWS_PER_W // G_CHUNK


def _sc_gather(W, idx2):
    mesh = plsc.VectorSubcoreMesh(core_axis_name="core",
                                  subcore_axis_name="subcore")

    @pl.kernel(out_type=jax.ShapeDtypeStruct((HROWS, DIM), W.dtype),
               mesh=mesh,
               scratch_types=[
                   pltpu.VMEM((1, HROWS_PER_W), jnp.int32),
                   pltpu.VMEM((G_CHUNK, DIM), W.dtype),
                   pltpu.VMEM((G_CHUNK, DIM), W.dtype),
                   pltpu.SemaphoreType.DMA,
                   pltpu.SemaphoreType.DMA,
               ])
    def gather_kernel(w_hbm, i_hbm, o_hbm, idx_v, g0, g1, gsem, osem):
        c = jax.lax.axis_index("core")
        s = jax.lax.axis_index("subcore")
        base = (c * 16 + s) * HROWS_PER_W
        pltpu.sync_copy(i_hbm.at[0, pl.ds(base, HROWS_PER_W)], idx_v.at[0])
        bufs = [g0, g1]
        outs = []
        for k in range(N_GCHUNK_H):
            gbuf = bufs[k % 2]
            if k >= 2:
                outs[k - 2].wait()
            pltpu.async_copy(
                w_hbm.at[idx_v.at[0, pl.ds(k * G_CHUNK, G_CHUNK)]],
                gbuf, gsem).wait()
            outs.append(pltpu.make_async_copy(
                gbuf, o_hbm.at[pl.ds(base + k * G_CHUNK, G_CHUNK), :], osem))
            outs[k].start()
        outs[N_GCHUNK_H - 2].wait()
        outs[N_GCHUNK_H - 1].wait()

    return gather_kernel(W, idx2)


def kernel(z, W):
    z_flat = z.reshape(ROWS, DIM)
    idx_pieces = []
    zq_pieces = []
    loss_sum = None
    for h in range(N_HALF):
        idx_h, loss_h = _distance_argmax(z_flat, W, h)
        zq_pieces.append(_sc_gather(W, idx_h))
        idx_pieces.append(idx_h)
        loss_sum = loss_h[0, 0] if loss_sum is None else loss_sum + loss_h[0, 0]
    loss = (1.0 + BETA) * loss_sum / (ROWS * DIM)
    if N_HALF == 1:
        z_q_st = zq_pieces[0].reshape(z.shape)
        encoding_indices = idx_pieces[0].reshape(z.shape[:-1])
    else:
        buf = jnp.zeros((ROWS, DIM), z.dtype)
        for h in range(N_HALF):
            buf = jax.lax.dynamic_update_slice(buf, zq_pieces[h],
                                               (h * HROWS, 0))
        z_q_st = buf.reshape(z.shape)
        encoding_indices = jnp.concatenate(idx_pieces, axis=1).reshape(
            z.shape[:-1])
    return (z_q_st, loss, encoding_indices)


# speedup vs baseline: 1.0359x; 1.0359x over previous
import jax
import jax.numpy as jnp
from jax.experimental import pallas as pl
from jax.experimental.pallas import tpu as pltpu
from jax.experimental.pallas import tpu_sc as plsc

NUM_CODES = 1024
DIM = 512
ROWS = 8192
BLK = 512
NBLK = ROWS // BLK
BETA = 0.25

R_SUB = 64
N_CHUNK = 128
N_BLOCK = 256

N_WORKERS = 32
ROWS_PER_W = ROWS // N_WORKERS
G_CHUNK = 64
N_GCHUNK = ROWS_PER_W // G_CHUNK


def _dist_kernel(z_ref, w_ref, idx_ref, loss_ref, acc_ref, ws_ref):
    b = pl.program_id(0)

    @pl.when(b == 0)
    def _():
        w0 = w_ref[...]
        ws_ref[0, :] = jnp.sum(w0 * w0, axis=1)
        acc_ref[0] = 0.0

    z = z_ref[...]
    z2 = z + z
    w = w_ref[...]
    mm2 = [
        jax.lax.dot_general(
            z2, w[i * N_BLOCK:(i + 1) * N_BLOCK, :],
            (((1,), (1,)), ((), ())),
            preferred_element_type=jnp.float32,
            precision=jax.lax.Precision.DEFAULT,
        )
        for i in range(NUM_CODES // N_BLOCK)
    ]
    rs = jnp.sum(z * z, axis=1, keepdims=True)
    nrs = -rs
    ws = ws_ref[0, :]

    part = None
    for r in range(BLK // R_SUB):
        rsl = slice(r * R_SUB, (r + 1) * R_SUB)
        nrb = nrs[rsl]
        m = None
        cid = None
        for c in range(NUM_CODES // N_CHUNK):
            csl = slice(c * N_CHUNK, (c + 1) * N_CHUNK)
            mm_tile = mm2[(c * N_CHUNK) // N_BLOCK]
            col = (c * N_CHUNK) % N_BLOCK
            t = nrb - ws[None, csl]
            d = t + mm_tile[rsl, col:col + N_CHUNK]
            if c == 0:
                m = d
                cid = jnp.zeros((R_SUB, N_CHUNK), jnp.int32)
            else:
                cond = d > m
                m = jnp.where(cond, d, m)
                cid = jnp.where(cond, c * N_CHUNK, cid)
        M = jnp.max(m, axis=1)
        lane = jax.lax.broadcasted_iota(jnp.int32, (R_SUB, N_CHUNK), 1)
        gid = cid + lane
        idxr = jnp.min(jnp.where(m == M[:, None], gid, NUM_CODES), axis=1)
        idx_ref[0, r * R_SUB:(r + 1) * R_SUB] = idxr
        psum = jnp.sum(-M)
        part = psum if part is None else part + psum
    acc_ref[0] += part

    @pl.when(b == pl.num_programs(0) - 1)
    def _():
        loss_ref[0, 0] = acc_ref[0]


N_HALF = 1
HROWS = ROWS // N_HALF
NBLK_H = HROWS // BLK


def _distance_argmax(z_flat, W, half):
    return pl.pallas_call(
        _dist_kernel,
        grid=(NBLK_H,),
        in_specs=[
            pl.BlockSpec((BLK, DIM), lambda b: (b + half * NBLK_H, 0)),
            pl.BlockSpec((NUM_CODES, DIM), lambda b: (0, 0)),
        ],
        out_specs=[
            pl.BlockSpec((1, BLK), lambda b: (0, b)),
            pl.BlockSpec(memory_space=pltpu.SMEM),
        ],
        out_shape=[
            jax.ShapeDtypeStruct((1, HROWS), jnp.int32),
            jax.ShapeDtypeStruct((1, 1), jnp.float32),
        ],
        scratch_shapes=[
            pltpu.SMEM((1,), jnp.float32),
            pltpu.VMEM((1, NUM_CODES), jnp.float32),
        ],
    )(z_flat, W)


HROWS_PER_W = HROWS // N_WORKERS
N_GCHUNK_H = HROWS_PER_W // G_CHUNK


def _sc_gather(W, idx2):
    mesh = plsc.VectorSubcoreMesh(core_axis_name="core",
                                  subcore_axis_name="subcore")

    @pl.kernel(out_type=jax.ShapeDtypeStruct((HROWS, DIM), W.dtype),
               mesh=mesh,
               scratch_types=[
                   pltpu.VMEM((1, HROWS_PER_W), jnp.int32),
                   pltpu.VMEM((G_CHUNK, DIM), W.dtype),
                   pltpu.VMEM((G_CHUNK, DIM), W.dtype),
                   pltpu.SemaphoreType.DMA,
                   pltpu.SemaphoreType.DMA,
               ])
    def gather_kernel(w_hbm, i_hbm, o_hbm, idx_v, g0, g1, gsem, osem):
        c = jax.lax.axis_index("core")
        s = jax.lax.axis_index("subcore")
        base = (c * 16 + s) * HROWS_PER_W
        pltpu.sync_copy(i_hbm.at[0, pl.ds(base, HROWS_PER_W)], idx_v.at[0])
        bufs = [g0, g1]
        n = N_GCHUNK_H

        def gath(k):
            return pltpu.make_async_copy(
                w_hbm.at[idx_v.at[0, pl.ds(k * G_CHUNK, G_CHUNK)]],
                bufs[k % 2], gsem)

        def outc(k):
            return pltpu.make_async_copy(
                bufs[k % 2],
                o_hbm.at[pl.ds(base + k * G_CHUNK, G_CHUNK), :], osem)

        gath(0).start()
        gath(1).start()
        for k in range(n):
            gath(k).wait()
            outc(k).start()
            if k + 2 < n:
                outc(k).wait()
                gath(k + 2).start()
        outc(n - 2).wait()
        outc(n - 1).wait()

    return gather_kernel(W, idx2)


def kernel(z, W):
    z_flat = z.reshape(ROWS, DIM)
    idx_pieces = []
    zq_pieces = []
    loss_sum = None
    for h in range(N_HALF):
        idx_h, loss_h = _distance_argmax(z_flat, W, h)
        zq_pieces.append(_sc_gather(W, idx_h))
        idx_pieces.append(idx_h)
        loss_sum = loss_h[0, 0] if loss_sum is None else loss_sum + loss_h[0, 0]
    loss = (1.0 + BETA) * loss_sum / (ROWS * DIM)
    if N_HALF == 1:
        z_q_st = zq_pieces[0].reshape(z.shape)
        encoding_indices = idx_pieces[0].reshape(z.shape[:-1])
    else:
        buf = jnp.zeros((ROWS, DIM), z.dtype)
        for h in range(N_HALF):
            buf = jax.lax.dynamic_update_slice(buf, zq_pieces[h],
                                               (h * HROWS, 0))
        z_q_st = buf.reshape(z.shape)
        encoding_indices = jnp.concatenate(idx_pieces, axis=1).reshape(
            z.shape[:-1])
    return (z_q_st, loss, encoding_indices)
